# Initial kernel scaffold; baseline (speedup 1.0000x reference)
import jax
import jax.numpy as jnp
from jax.experimental import pallas as pl


def kernel(x, edge_index, edge_attr, W_ih_n, W_hh_n, b_ih_n, b_hh_n, W_ih_e, W_hh_e, b_ih_e, b_hh_e, fc1_W, fc1_b, node_mpn_W, node_mpn_b, edge_mpn_W, edge_mpn_b, pred_W, pred_b):
    raise NotImplementedError("write your pallas kernel here")


# speedup vs baseline: 2.3311x; 2.3311x over previous
import functools

import jax
import jax.numpy as jnp
from jax import lax
from jax.experimental import pallas as pl
from jax.experimental.pallas import tpu as pltpu
from jax.experimental.pallas import tpu_sc as plsc

H = 128
N_NODES = 10000
N_EDGES = 160000
NUM_ITER = 20
NP = 10240
NW = 32
CHUNK = 128
NCHUNK = 40
EP = NW * NCHUNK * CHUNK
BN = 1024
BE = 2048
STRIPE = NP // 16


def _lrelu(v):
    return jnp.maximum(v, 0.01 * v)


def _dot(a, b):
    return jnp.dot(a, b, preferred_element_type=jnp.float32)


def _lstm_gates(g, c_prev):
    i = jax.nn.sigmoid(g[:, :H])
    f = jax.nn.sigmoid(g[:, H:2 * H])
    gg = jnp.tanh(g[:, 2 * H:3 * H])
    o = jax.nn.sigmoid(g[:, 3 * H:])
    c2 = f * c_prev + i * gg
    return o * jnp.tanh(c2), c2


def _node_tail(ni, h_prev, c_prev, whh, wih, b, wcat, nb, pb, out_prev,
               h2o, c2o, ps_o, pd_o, pnh_o, out_o):
    g = _dot(ni, wih[...]) + b[...]
    if h_prev is not None:
        g = g + _dot(h_prev, whh[...])
        h2, c2 = _lstm_gates(g, c_prev)
    else:
        h2, c2 = _lstm_gates(g, 0.0)
    pr = _dot(h2, wcat[...])
    h2o[...] = h2
    c2o[...] = c2
    ps_o[...] = pr[:, :20]
    pd_o[...] = pr[:, 20:40]
    pnh_o[...] = pr[:, 40:56] + nb[...]
    acc = pr[:, 56:60] + pb[...]
    out_o[...] = acc if out_prev is None else out_prev + acc


def _node_body0(x_ref, wih, b, wcat, nb, pb, h2o, c2o, ps_o, pd_o, pnh_o, out_o):
    _node_tail(x_ref[...], None, None, None, wih, b, wcat, nb, pb, None,
               h2o, c2o, ps_o, pd_o, pnh_o, out_o)


def _node_body1(s2, cnt2, pnh, hp, cp, wih, whh, b, wcat, nb, pb, op,
                h2o, c2o, ps_o, pd_o, pnh_o, out_o):
    s = s2[0] + s2[1]
    cnt = jnp.maximum(cnt2[0, :, :1] + cnt2[1, :, :1], 1.0)
    ni = _lrelu(s / cnt + pnh[...])
    _node_tail(ni, hp[...], cp[...], whh, wih, b, wcat, nb, pb, op[...],
               h2o, c2o, ps_o, pd_o, pnh_o, out_o)


def _w_spec(shape):
    return pl.BlockSpec(shape, lambda i: (0,) * len(shape))


_NODE_OUT = (
    jax.ShapeDtypeStruct((NP, H), jnp.float32),
    jax.ShapeDtypeStruct((NP, H), jnp.float32),
    jax.ShapeDtypeStruct((NP, 20), jnp.float32),
    jax.ShapeDtypeStruct((NP, 20), jnp.float32),
    jax.ShapeDtypeStruct((NP, 16), jnp.float32),
    jax.ShapeDtypeStruct((NP, 4), jnp.float32),
)

_NODE_OUT_SPECS = [
    pl.BlockSpec((BN, H), lambda i: (i, 0)),
    pl.BlockSpec((BN, H), lambda i: (i, 0)),
    pl.BlockSpec((BN, 20), lambda i: (i, 0)),
    pl.BlockSpec((BN, 20), lambda i: (i, 0)),
    pl.BlockSpec((BN, 16), lambda i: (i, 0)),
    pl.BlockSpec((BN, 4), lambda i: (i, 0)),
]


def _node_call0(x, wih, b, wcat, nb, pb):
    return pl.pallas_call(
        _node_body0,
        grid=(NP // BN,),
        in_specs=[
            pl.BlockSpec((BN, 16), lambda i: (i, 0)),
            _w_spec((16, 4 * H)),
            _w_spec((1, 4 * H)),
            _w_spec((H, 60)),
            _w_spec((1, 16)),
            _w_spec((1, 4)),
        ],
        out_specs=_NODE_OUT_SPECS,
        out_shape=_NODE_OUT,
    )(x, wih, b, wcat, nb, pb)


def _node_call1(s2, cnt2, pnh, hp, cp, wih, whh, b, wcat, nb, pb, op):
    return pl.pallas_call(
        _node_body1,
        grid=(NP // BN,),
        in_specs=[
            pl.BlockSpec((2, BN, 16), lambda i: (0, i, 0)),
            pl.BlockSpec((2, BN, 16), lambda i: (0, i, 0)),
            pl.BlockSpec((BN, 16), lambda i: (i, 0)),
            pl.BlockSpec((BN, H), lambda i: (i, 0)),
            pl.BlockSpec((BN, H), lambda i: (i, 0)),
            _w_spec((16, 4 * H)),
            _w_spec((H, 4 * H)),
            _w_spec((1, 4 * H)),
            _w_spec((H, 60)),
            _w_spec((1, 16)),
            _w_spec((1, 4)),
            pl.BlockSpec((BN, 4), lambda i: (i, 0)),
        ],
        out_specs=_NODE_OUT_SPECS,
        out_shape=_NODE_OUT,
    )(s2, cnt2, pnh, hp, cp, wih, whh, b, wcat, nb, pb, op)


def _edge_tail(ei, h_prev, c_prev, whh, wih, b, wcat, eb,
               h2o, c2o, ee_o, q_o):
    g = _dot(ei, wih[...]) + b[...]
    if h_prev is not None:
        g = g + _dot(h_prev, whh[...])
        h2, c2 = _lstm_gates(g, c_prev)
    else:
        h2, c2 = _lstm_gates(g, 0.0)
    pr = _dot(h2, wcat[...])
    h2o[...] = h2
    c2o[...] = c2
    ee_o[...] = pr[:, :20] + eb[...]
    q_o[...] = pr[:, 20:36]


def _edge_body0(ei_ref, wih, b, wcat, eb, h2o, c2o, ee_o, q_o):
    _edge_tail(ei_ref[...], None, None, None, wih, b, wcat, eb,
               h2o, c2o, ee_o, q_o)


def _edge_body1(rs, rd, eep, hp, cp, wih, whh, b, wcat, eb,
                h2o, c2o, ee_o, q_o):
    ei = _lrelu(rs[...] + rd[...] + eep[...])
    _edge_tail(ei, hp[...], cp[...], whh, wih, b, wcat, eb,
               h2o, c2o, ee_o, q_o)


_EDGE_OUT = (
    jax.ShapeDtypeStruct((EP, H), jnp.float32),
    jax.ShapeDtypeStruct((EP, H), jnp.float32),
    jax.ShapeDtypeStruct((EP, 20), jnp.float32),
    jax.ShapeDtypeStruct((EP, 16), jnp.float32),
)

_EDGE_OUT_SPECS = [
    pl.BlockSpec((BE, H), lambda i: (i, 0)),
    pl.BlockSpec((BE, H), lambda i: (i, 0)),
    pl.BlockSpec((BE, 20), lambda i: (i, 0)),
    pl.BlockSpec((BE, 16), lambda i: (i, 0)),
]


def _edge_call0(ei, wih, b, wcat, eb):
    return pl.pallas_call(
        _edge_body0,
        grid=(EP // BE,),
        in_specs=[
            pl.BlockSpec((BE, 20), lambda i: (i, 0)),
            _w_spec((20, 4 * H)),
            _w_spec((1, 4 * H)),
            _w_spec((H, 36)),
            _w_spec((1, 20)),
        ],
        out_specs=_EDGE_OUT_SPECS,
        out_shape=_EDGE_OUT,
    )(ei, wih, b, wcat, eb)


def _edge_call1(rs, rd, eep, hp, cp, wih, whh, b, wcat, eb):
    return pl.pallas_call(
        _edge_body1,
        grid=(EP // BE,),
        in_specs=[
            pl.BlockSpec((BE, 20), lambda i: (i, 0)),
            pl.BlockSpec((BE, 20), lambda i: (i, 0)),
            pl.BlockSpec((BE, 20), lambda i: (i, 0)),
            pl.BlockSpec((BE, H), lambda i: (i, 0)),
            pl.BlockSpec((BE, H), lambda i: (i, 0)),
            _w_spec((20, 4 * H)),
            _w_spec((H, 4 * H)),
            _w_spec((1, 4 * H)),
            _w_spec((H, 36)),
            _w_spec((1, 20)),
        ],
        out_specs=_EDGE_OUT_SPECS,
        out_shape=_EDGE_OUT,
    )(rs, rd, eep, hp, cp, wih, whh, b, wcat, eb)


def _ef_body(ea, fw, fb, ef_o):
    ef_o[...] = _dot(ea[...], fw[...]) + fb[...]


def _ef_call(ea, fw, fb):
    return pl.pallas_call(
        _ef_body,
        grid=(EP // BE,),
        in_specs=[
            pl.BlockSpec((BE, 2), lambda i: (i, 0)),
            _w_spec((2, 20)),
            _w_spec((1, 20)),
        ],
        out_specs=pl.BlockSpec((BE, 20), lambda i: (i, 0)),
        out_shape=jax.ShapeDtypeStruct((EP, 20), jnp.float32),
    )(ea, fw, fb)


def _sc_mesh():
    return plsc.VectorSubcoreMesh(core_axis_name="c", subcore_axis_name="s",
                                  num_cores=2, num_subcores=16)


@functools.cache
def _build_sc_combine():
    @functools.partial(
        pl.kernel,
        out_type=(
            jax.ShapeDtypeStruct((EP, 20), jnp.float32),
            jax.ShapeDtypeStruct((EP, 20), jnp.float32),
            jax.ShapeDtypeStruct((2, NP, 16), jnp.float32),
        ),
        mesh=_sc_mesh(),
        scratch_types=[
            pltpu.VMEM((NCHUNK, CHUNK), jnp.int32),
            pltpu.VMEM((NCHUNK, CHUNK), jnp.int32),
            pltpu.VMEM((CHUNK, 20), jnp.float32),
            pltpu.VMEM((CHUNK, 20), jnp.float32),
            pltpu.VMEM((CHUNK, 16), jnp.float32),
            pltpu.VMEM((STRIPE, 16), jnp.float32),
            pltpu.VMEM_SHARED((NP, 16), jnp.float32),
            pltpu.SemaphoreType.DMA,
            pltpu.SemaphoreType.DMA,
        ],
        compiler_params=pltpu.CompilerParams(use_tc_tiling_on_sc=False),
    )
    def _sc_combine(ps_hbm, pd_hbm, q_hbm, src_hbm, dst_hbm,
                    rs_hbm, rd_hbm, s2_hbm,
                    idx_s, idx_d, buf_s, buf_d, buf_q, zbuf, s_sh,
                    sem_s, sem_d):
        cid = lax.axis_index("c")
        sid = lax.axis_index("s")
        wid = sid * 2 + cid
        base = wid * (NCHUNK * CHUNK)
        pltpu.sync_copy(src_hbm.at[wid], idx_s)
        pltpu.sync_copy(dst_hbm.at[wid], idx_d)

        def zloop(i, carry):
            zbuf[i] = jnp.zeros((16,), jnp.float32)
            return carry

        lax.fori_loop(0, STRIPE, zloop, 0)
        pltpu.sync_copy(zbuf, s_sh.at[pl.ds(sid * STRIPE, STRIPE)])
        plsc.subcore_barrier()

        def body(j, carry):
            cp_s = pltpu.async_copy(ps_hbm.at[idx_s.at[j]], buf_s, sem_s)
            cp_d = pltpu.async_copy(pd_hbm.at[idx_d.at[j]], buf_d, sem_d)
            pltpu.sync_copy(q_hbm.at[pl.ds(base + j * CHUNK, CHUNK)], buf_q)
            pltpu.sync_copy(buf_q, s_sh.at[idx_d.at[j]], add=True)
            cp_s.wait()
            cp_d.wait()
            pltpu.sync_copy(buf_s, rs_hbm.at[pl.ds(base + j * CHUNK, CHUNK)])
            pltpu.sync_copy(buf_d, rd_hbm.at[pl.ds(base + j * CHUNK, CHUNK)])
            return carry

        lax.fori_loop(0, NCHUNK, body, 0)
        plsc.subcore_barrier()
        pltpu.sync_copy(s_sh.at[pl.ds(sid * STRIPE, STRIPE)],
                        s2_hbm.at[cid, pl.ds(sid * STRIPE, STRIPE)])

    return _sc_combine


@functools.cache
def _build_sc_counts():
    @functools.partial(
        pl.kernel,
        out_type=jax.ShapeDtypeStruct((2, NP, 16), jnp.float32),
        mesh=_sc_mesh(),
        scratch_types=[
            pltpu.VMEM((NCHUNK, CHUNK), jnp.int32),
            pltpu.VMEM((CHUNK, 16), jnp.float32),
            pltpu.VMEM((STRIPE, 16), jnp.float32),
            pltpu.VMEM_SHARED((NP, 16), jnp.float32),
        ],
        compiler_params=pltpu.CompilerParams(use_tc_tiling_on_sc=False),
    )
    def _sc_counts(dst_hbm, c2_hbm, idx_d, obuf, zbuf, s_sh):
        cid = lax.axis_index("c")
        sid = lax.axis_index("s")
        wid = sid * 2 + cid
        pltpu.sync_copy(dst_hbm.at[wid], idx_d)

        def floop(i, carry):
            obuf[i] = jnp.ones((16,), jnp.float32)
            return carry

        lax.fori_loop(0, CHUNK, floop, 0)

        def zloop(i, carry):
            zbuf[i] = jnp.zeros((16,), jnp.float32)
            return carry

        lax.fori_loop(0, STRIPE, zloop, 0)
        pltpu.sync_copy(zbuf, s_sh.at[pl.ds(sid * STRIPE, STRIPE)])
        plsc.subcore_barrier()

        def body(j, carry):
            pltpu.sync_copy(obuf, s_sh.at[idx_d.at[j]], add=True)
            return carry

        lax.fori_loop(0, NCHUNK, body, 0)
        plsc.subcore_barrier()
        pltpu.sync_copy(s_sh.at[pl.ds(sid * STRIPE, STRIPE)],
                        c2_hbm.at[cid, pl.ds(sid * STRIPE, STRIPE)])

    return _sc_counts


def _sc_combine_call(ps, pd, q, src3, dst3):
    return _build_sc_combine()(ps, pd, q, src3, dst3)


def _sc_counts_call(dst3):
    return _build_sc_counts()(dst3)


def kernel(x, edge_index, edge_attr, W_ih_n, W_hh_n, b_ih_n, b_hh_n,
           W_ih_e, W_hh_e, b_ih_e, b_hh_e, fc1_W, fc1_b,
           node_mpn_W, node_mpn_b, edge_mpn_W, edge_mpn_b, pred_W, pred_b):
    f32 = jnp.float32
    W_es = edge_mpn_W[:, :H].T
    W_ee = edge_mpn_W[:, H:2 * H].T
    W_ed = edge_mpn_W[:, 2 * H:].T
    W_na = node_mpn_W[:, :H].T
    W_nh = node_mpn_W[:, H:].T
    wcat_n = jnp.concatenate([
        jnp.broadcast_to(W_es[None], (NUM_ITER, H, 20)),
        jnp.broadcast_to(W_ed[None], (NUM_ITER, H, 20)),
        jnp.broadcast_to(W_nh[None], (NUM_ITER, H, 16)),
        jnp.transpose(pred_W, (0, 2, 1)),
    ], axis=2)
    wcat_e = jnp.concatenate([W_ee, W_na], axis=1)
    wih_n = W_ih_n.T
    whh_n = W_hh_n.T
    wih_e = W_ih_e.T
    whh_e = W_hh_e.T
    b_n = (b_ih_n + b_hh_n)[None]
    b_e = (b_ih_e + b_hh_e)[None]
    nb = node_mpn_b[None]
    eb = edge_mpn_b[None]
    fw = fc1_W.T
    fb = fc1_b[None]

    x_p = jnp.pad(x, ((0, NP - N_NODES), (0, 0)))
    ea_p = jnp.pad(edge_attr, ((0, EP - N_EDGES), (0, 0)))
    pad_idx = jnp.full((EP - N_EDGES,), N_NODES, jnp.int32)
    src3 = jnp.concatenate([edge_index[0], pad_idx]).reshape(NW, NCHUNK, CHUNK)
    dst3 = jnp.concatenate([edge_index[1], pad_idx]).reshape(NW, NCHUNK, CHUNK)

    cnt2 = _sc_counts_call(dst3)
    ef = _ef_call(ea_p, fw, fb)

    h_n, c_n, ps, pd_, pnh, out = _node_call0(
        x_p, wih_n, b_n, wcat_n[0], nb, pred_b[0][None])
    h_e, c_e, ee, q = _edge_call0(ef, wih_e, b_e, wcat_e, eb)
    rs, rd, s2 = _sc_combine_call(ps, pd_, q, src3, dst3)

    for it in range(1, NUM_ITER):
        h_n, c_n, ps, pd_, pnh, out = _node_call1(
            s2, cnt2, pnh, h_n, c_n, wih_n, whh_n, b_n,
            wcat_n[it], nb, pred_b[it][None], out)
        h_e, c_e, ee, q = _edge_call1(
            rs, rd, ee, h_e, c_e, wih_e, whh_e, b_e, wcat_e, eb)
        if it < NUM_ITER - 1:
            rs, rd, s2 = _sc_combine_call(ps, pd_, q, src3, dst3)

    return out[:N_NODES]
